# baseline (speedup 1.0000x reference)
import functools

import jax
import jax.numpy as jnp
from jax.experimental import pallas as pl

B = 65536
OBS = 128
H = 256
D = 32
K = 512

BM = 16384


def _vqvae_block(obs_ref, ew1_ref, eb1_ref, ew2_ref, eb2_ref, ew3_ref, eb3_ref,
                 cb_ref, cbm2_ref, c2_ref, iota_ref, dw1_ref, db1_ref, dw2_ref,
                 db2_ref, dw3_ref, db3_ref, out_ref):
    x = obs_ref[...]
    h = jax.nn.relu(jnp.dot(x, ew1_ref[...], preferred_element_type=jnp.float32)
                    + eb1_ref[...])
    h = jax.nn.relu(jnp.dot(h, ew2_ref[...], preferred_element_type=jnp.float32)
                    + eb2_ref[...])
    z = jnp.dot(h, ew3_ref[...], preferred_element_type=jnp.float32) + eb3_ref[...]

    cross2 = jax.lax.dot_general(z, cbm2_ref[...], (((1,), (1,)), ((), ())),
                                 preferred_element_type=jnp.float32)
    dists = c2_ref[...] + cross2
    m = jnp.min(dists, axis=1, keepdims=True)
    iota = iota_ref[...]
    idx = jnp.min(jnp.where(dists == m, iota, float(K)), axis=1, keepdims=True)
    onehot = jnp.where(iota == idx, 1.0, 0.0)
    q = jnp.dot(onehot, cb_ref[...], preferred_element_type=jnp.float32)

    h = jax.nn.relu(jnp.dot(q, dw1_ref[...], preferred_element_type=jnp.float32)
                    + db1_ref[...])
    h = jax.nn.relu(jnp.dot(h, dw2_ref[...], preferred_element_type=jnp.float32)
                    + db2_ref[...])
    out_ref[...] = (jnp.dot(h, dw3_ref[...], preferred_element_type=jnp.float32)
                    + db3_ref[...])


@jax.jit
def kernel(observations, enc_w1, enc_b1, enc_w2, enc_b2, enc_w3, enc_b3,
           codebook, dec_w1, dec_b1, dec_w2, dec_b2, dec_w3, dec_b3):
    def rep(shape):
        return pl.BlockSpec(shape, lambda i: (0,) * len(shape))

    grid = (B // BM,)
    return pl.pallas_call(
        _vqvae_block,
        grid=grid,
        in_specs=[
            pl.BlockSpec((BM, OBS), lambda i: (i, 0)),
            rep((OBS, H)), rep((1, H)),
            rep((H, H)), rep((1, H)),
            rep((H, D)), rep((1, D)),
            rep((K, D)), rep((K, D)), rep((1, K)), rep((1, K)),
            rep((D, H)), rep((1, H)),
            rep((H, H)), rep((1, H)),
            rep((H, OBS)), rep((1, OBS)),
        ],
        out_specs=pl.BlockSpec((BM, OBS), lambda i: (i, 0)),
        out_shape=jax.ShapeDtypeStruct((B, OBS), jnp.float32),
    )(observations,
      enc_w1, enc_b1[None, :], enc_w2, enc_b2[None, :], enc_w3, enc_b3[None, :],
      codebook, -2.0 * codebook,
      jnp.sum(codebook * codebook, axis=1)[None, :],
      jnp.arange(K, dtype=jnp.float32)[None, :],
      dec_w1, dec_b1[None, :], dec_w2, dec_b2[None, :], dec_w3, dec_b3[None, :])


# speedup vs baseline: 1.0014x; 1.0014x over previous
import functools

import jax
import jax.numpy as jnp
from jax.experimental import pallas as pl
from jax.experimental.pallas import tpu as pltpu

B = 65536
OBS = 128
H = 256
D = 32
K = 512

BM = 8192


def _vqvae_block(obs_ref, ew1_ref, eb1_ref, ew2_ref, eb2_ref, ew3_ref, eb3_ref,
                 cb_ref, cbm2_ref, c2_ref, iota_ref, dw1_ref, db1_ref, dw2_ref,
                 db2_ref, dw3_ref, db3_ref, out_ref):
    x = obs_ref[...]
    h = jax.nn.relu(jnp.dot(x, ew1_ref[...], preferred_element_type=jnp.float32)
                    + eb1_ref[...])
    h = jax.nn.relu(jnp.dot(h, ew2_ref[...], preferred_element_type=jnp.float32)
                    + eb2_ref[...])
    z = jnp.dot(h, ew3_ref[...], preferred_element_type=jnp.float32) + eb3_ref[...]

    cross2 = jax.lax.dot_general(z, cbm2_ref[...], (((1,), (1,)), ((), ())),
                                 preferred_element_type=jnp.float32)
    dists = c2_ref[...] + cross2
    m = jnp.min(dists, axis=1, keepdims=True)
    iota = iota_ref[...]
    idx = jnp.min(jnp.where(dists == m, iota, float(K)), axis=1, keepdims=True)
    onehot = jnp.where(iota == idx, 1.0, 0.0)
    q = jnp.dot(onehot, cb_ref[...], preferred_element_type=jnp.float32)

    h = jax.nn.relu(jnp.dot(q, dw1_ref[...], preferred_element_type=jnp.float32)
                    + db1_ref[...])
    h = jax.nn.relu(jnp.dot(h, dw2_ref[...], preferred_element_type=jnp.float32)
                    + db2_ref[...])
    out_ref[...] = (jnp.dot(h, dw3_ref[...], preferred_element_type=jnp.float32)
                    + db3_ref[...])


@jax.jit
def kernel(observations, enc_w1, enc_b1, enc_w2, enc_b2, enc_w3, enc_b3,
           codebook, dec_w1, dec_b1, dec_w2, dec_b2, dec_w3, dec_b3):
    def rep(shape):
        return pl.BlockSpec(shape, lambda i: (0,) * len(shape))

    grid = (B // BM,)
    return pl.pallas_call(
        _vqvae_block,
        grid=grid,
        in_specs=[
            pl.BlockSpec((BM, OBS), lambda i: (i, 0)),
            rep((OBS, H)), rep((1, H)),
            rep((H, H)), rep((1, H)),
            rep((H, D)), rep((1, D)),
            rep((K, D)), rep((K, D)), rep((1, K)), rep((1, K)),
            rep((D, H)), rep((1, H)),
            rep((H, H)), rep((1, H)),
            rep((H, OBS)), rep((1, OBS)),
        ],
        compiler_params=pltpu.CompilerParams(
            dimension_semantics=("parallel",)),
        out_specs=pl.BlockSpec((BM, OBS), lambda i: (i, 0)),
        out_shape=jax.ShapeDtypeStruct((B, OBS), jnp.float32),
    )(observations,
      enc_w1, enc_b1[None, :], enc_w2, enc_b2[None, :], enc_w3, enc_b3[None, :],
      codebook, -2.0 * codebook,
      jnp.sum(codebook * codebook, axis=1)[None, :],
      jnp.arange(K, dtype=jnp.float32)[None, :],
      dec_w1, dec_b1[None, :], dec_w2, dec_b2[None, :], dec_w3, dec_b3[None, :])
